# baseline (speedup 1.0000x reference)
import functools

import jax
import jax.numpy as jnp
from jax import lax
from jax.experimental import pallas as pl
from jax.experimental.pallas import tpu as pltpu
from jax.experimental.pallas import tpu_sc as plsc

NC = 2
NS = 16
NW = NC * NS
EB = 128
NSTAGE = 4


def _ceil_to(a, m):
    return (a + m - 1) // m * m


def _deg_body(steps, dst_hbm, zeros_hbm, out_hbm, deg_sh, didx_v, ones_v, sem):
    npad = deg_sh.shape[0]
    c = lax.axis_index("c")
    s = lax.axis_index("s")
    rows = npad // NS
    base_r = s * rows
    pltpu.sync_copy(zeros_hbm.at[pl.ds(base_r, rows)], deg_sh.at[pl.ds(base_r, rows)])
    def fill(k, carry):
        ones_v[pl.ds(k * 16, 16)] = jnp.ones((16,), jnp.float32)
        return carry
    lax.fori_loop(0, EB // 16, fill, 0)
    w = s * NC + c
    pltpu.sync_copy(dst_hbm.at[w], didx_v)
    plsc.subcore_barrier()

    def body(i, carry):
        for b in range(4):
            pltpu.async_copy(ones_v, deg_sh.at[didx_v.at[i * 4 + b]],
                             sem, add=True)
        for b in range(4):
            pltpu.make_async_copy(ones_v, deg_sh.at[didx_v.at[i * 4 + b]],
                                  sem).wait()
        return carry
    lax.fori_loop(0, steps // 4, body, 0)
    plsc.subcore_barrier()
    pltpu.sync_copy(deg_sh.at[pl.ds(base_r, rows)], out_hbm.at[c, pl.ds(base_r, rows)])


def _agg_body(steps, g_hbm, src_hbm, dst_hbm, zeros_hbm, out_hbm,
              acc_sh, sidxA, sidxB, didxA, didxB,
              rows0, rows1, gsem0, gsem1, isem):
    npad, d = acc_sh.shape
    c = lax.axis_index("c")
    s = lax.axis_index("s")
    rows_pt = npad // NS
    base_r = s * rows_pt
    rows = (rows0, rows1)
    gsems = (gsem0, gsem1)
    sidx = (sidxA, sidxB)
    didx = (didxA, didxB)
    T = steps // NSTAGE

    @pl.when(c == 0)
    def _():
        pltpu.sync_copy(g_hbm.at[pl.ds(base_r, rows_pt)], acc_sh.at[pl.ds(base_r, rows_pt)])
    @pl.when(c != 0)
    def _():
        pltpu.sync_copy(zeros_hbm.at[pl.ds(base_r, rows_pt)], acc_sh.at[pl.ds(base_r, rows_pt)])

    w = s * NC + c
    pltpu.sync_copy(src_hbm.at[w, 0], sidx[0])
    pltpu.sync_copy(dst_hbm.at[w, 0], didx[0])

    H = EB // 2

    def fire_gather(q, r, b):
        idx = sidx[q % 2].at[r]
        pltpu.async_copy(g_hbm.at[idx.at[pl.ds(0, H)]], rows[b].at[pl.ds(0, H)], gsems[b])
        pltpu.async_copy(g_hbm.at[idx.at[pl.ds(H, H)]], rows[b].at[pl.ds(H, H)], gsems[b])

    def wait_gather(b):
        pltpu.make_async_copy(g_hbm.at[pl.ds(0, EB)], rows[b], gsems[b]).wait()

    def scatter(q, r, b):
        pltpu.sync_copy(rows[b], acc_sh.at[didx[q % 2].at[r]], add=True)

    fire_gather(0, 0, 0)
    fire_gather(0, 1, 1)
    plsc.subcore_barrier()

    for q in range(NSTAGE):
        for j in range(4):
            b = j % 2
            wait_gather(b)
            scatter(q, j, b)
            fire_gather(q, j + 2, b)
        if q + 1 < NSTAGE:
            nb = (q + 1) % 2
            pltpu.async_copy(src_hbm.at[w, q + 1], sidx[nb], isem)
            pltpu.async_copy(dst_hbm.at[w, q + 1], didx[nb], isem)

        def body(i, carry):
            for pb in range(2):
                j = 2 * i + pb
                wait_gather(pb)
                scatter(q, j, pb)
                fire_gather(q, j + 2, pb)
            return carry
        lax.fori_loop(2, T // 2 - 1, body, 0)

        if q + 1 < NSTAGE:
            nb = (q + 1) % 2
            pltpu.make_async_copy(src_hbm.at[w, q + 1], sidx[nb], isem).wait()
            pltpu.make_async_copy(dst_hbm.at[w, q + 1], didx[nb], isem).wait()
        for r, j in enumerate(range(T - 2, T)):
            b = j % 2
            wait_gather(b)
            scatter(q, j, b)
            if q + 1 < NSTAGE:
                fire_gather(q + 1, r, b)

    plsc.subcore_barrier()
    pltpu.sync_copy(acc_sh.at[pl.ds(base_r, rows_pt)],
                    out_hbm.at[c, pl.ds(base_r, rows_pt)])


def _dinv_body(degs_ref, out_ref):
    d = degs_ref[0, :] + degs_ref[1, :] + 1.0
    out_ref[...] = lax.rsqrt(d)


def _mm_first_body(x_ref, w_ref, dinv_ref, g_ref):
    h = jnp.dot(x_ref[...], w_ref[...], preferred_element_type=jnp.float32)
    g_ref[...] = h * dinv_ref[...]


def _mm_mid_body(p0_ref, p1_ref, b_ref, dinv_ref, w_ref, g_ref):
    z = jnp.maximum((p0_ref[...] + p1_ref[...]) * dinv_ref[...] + b_ref[...], 0.0)
    h = jnp.dot(z, w_ref[...], preferred_element_type=jnp.float32)
    g_ref[...] = h * dinv_ref[...]


def _mm_last_body(p0_ref, p1_ref, b_ref, dinv_ref, out_ref):
    out_ref[...] = (p0_ref[...] + p1_ref[...]) * dinv_ref[...] + b_ref[...]


def kernel(x, edge_index, W1, b1, W2, b2, W3, b3):
    n, d = x.shape
    e = edge_index.shape[1]
    npad = _ceil_to(n + 1, 2048)
    epad = _ceil_to(e, NW * EB * NSTAGE * 2)
    steps = epad // (NW * EB)
    R = 2048

    src = jnp.concatenate([edge_index[0], jnp.zeros((epad - e,), jnp.int32)])
    dst = jnp.concatenate([edge_index[1], jnp.full((epad - e,), n, jnp.int32)])
    dst_flat = dst.reshape(NW, steps, EB)
    src = src.reshape(NW, NSTAGE, steps // NSTAGE, EB)
    dst = dst.reshape(NW, NSTAGE, steps // NSTAGE, EB)
    xp = jnp.pad(x, ((0, npad - n), (0, 0)))
    zeros2d = jnp.zeros((npad, d), jnp.float32)
    zeros1d = jnp.zeros((npad,), jnp.float32)

    mesh = plsc.VectorSubcoreMesh(core_axis_name="c", subcore_axis_name="s",
                                  num_cores=NC, num_subcores=NS)

    deg_call = pl.kernel(
        functools.partial(_deg_body, steps),
        out_type=jax.ShapeDtypeStruct((NC, npad), jnp.float32),
        mesh=mesh,
        scratch_types=[
            pltpu.VMEM_SHARED((npad,), jnp.float32),
            pltpu.VMEM((steps, EB), jnp.int32),
            pltpu.VMEM((EB,), jnp.float32),
            pltpu.SemaphoreType.DMA,
        ],
    )

    agg_call = pl.kernel(
        functools.partial(_agg_body, steps),
        out_type=jax.ShapeDtypeStruct((NC, npad, d), jnp.float32),
        mesh=mesh,
        scratch_types=[
            pltpu.VMEM_SHARED((npad, d), jnp.float32),
            pltpu.VMEM((steps // NSTAGE, EB), jnp.int32),
            pltpu.VMEM((steps // NSTAGE, EB), jnp.int32),
            pltpu.VMEM((steps // NSTAGE, EB), jnp.int32),
            pltpu.VMEM((steps // NSTAGE, EB), jnp.int32),
            pltpu.VMEM((EB, d), jnp.float32),
            pltpu.VMEM((EB, d), jnp.float32),
            pltpu.SemaphoreType.DMA,
            pltpu.SemaphoreType.DMA,
            pltpu.SemaphoreType.DMA,
        ],
    )

    dinv_call = pl.pallas_call(
        _dinv_body,
        out_shape=jax.ShapeDtypeStruct((npad,), jnp.float32),
    )

    row_spec = pl.BlockSpec((R, d), lambda i: (i, 0))
    w_spec = pl.BlockSpec((d, d), lambda i: (0, 0))
    b_spec = pl.BlockSpec((1, d), lambda i: (0, 0))
    dinv_spec = pl.BlockSpec((R, 1), lambda i: (i, 0))
    grid = (npad // R,)
    gshape = jax.ShapeDtypeStruct((npad, d), jnp.float32)

    mm_first = pl.pallas_call(
        _mm_first_body, grid=grid,
        in_specs=[row_spec, w_spec, dinv_spec],
        out_specs=row_spec, out_shape=gshape,
    )
    mm_mid = pl.pallas_call(
        _mm_mid_body, grid=grid,
        in_specs=[row_spec, row_spec, b_spec, dinv_spec, w_spec],
        out_specs=row_spec, out_shape=gshape,
    )
    mm_last = pl.pallas_call(
        _mm_last_body, grid=grid,
        in_specs=[row_spec, row_spec, b_spec, dinv_spec],
        out_specs=row_spec, out_shape=gshape,
    )

    degs = deg_call(dst_flat, zeros1d)
    dinv_col = dinv_call(degs).reshape(npad, 1)
    b1r = b1.reshape(1, d)
    b2r = b2.reshape(1, d)
    b3r = b3.reshape(1, d)

    g = mm_first(xp, W1, dinv_col)
    p = agg_call(g, src, dst, zeros2d)
    g = mm_mid(p[0], p[1], b1r, dinv_col, W2)
    p = agg_call(g, src, dst, zeros2d)
    g = mm_mid(p[0], p[1], b2r, dinv_col, W3)
    p = agg_call(g, src, dst, zeros2d)
    out = mm_last(p[0], p[1], b3r, dinv_col)
    return out[:n]


# speedup vs baseline: 1.0047x; 1.0047x over previous
import functools

import jax
import jax.numpy as jnp
from jax import lax
from jax.experimental import pallas as pl
from jax.experimental.pallas import tpu as pltpu
from jax.experimental.pallas import tpu_sc as plsc

NC = 2
NS = 16
NW = NC * NS
EB = 128
NSTAGE = 4


def _ceil_to(a, m):
    return (a + m - 1) // m * m


def _deg_body(steps, dst_hbm, zeros_hbm, out_hbm, deg_sh, didx_v, ones_v, sem):
    npad = deg_sh.shape[0]
    c = lax.axis_index("c")
    s = lax.axis_index("s")
    rows = npad // NS
    base_r = s * rows
    pltpu.sync_copy(zeros_hbm.at[pl.ds(base_r, rows)], deg_sh.at[pl.ds(base_r, rows)])
    def fill(k, carry):
        ones_v[pl.ds(k * 16, 16)] = jnp.ones((16,), jnp.float32)
        return carry
    lax.fori_loop(0, EB // 16, fill, 0)
    w = s * NC + c
    pltpu.sync_copy(dst_hbm.at[w], didx_v)
    plsc.subcore_barrier()

    def body(i, carry):
        for b in range(4):
            pltpu.async_copy(ones_v, deg_sh.at[didx_v.at[i * 4 + b]],
                             sem, add=True)
        for b in range(4):
            pltpu.make_async_copy(ones_v, deg_sh.at[didx_v.at[i * 4 + b]],
                                  sem).wait()
        return carry
    lax.fori_loop(0, steps // 4, body, 0)
    plsc.subcore_barrier()
    pltpu.sync_copy(deg_sh.at[pl.ds(base_r, rows)], out_hbm.at[c, pl.ds(base_r, rows)])


def _agg_body(steps, g_hbm, src_hbm, dst_hbm, zeros_hbm, out_hbm,
              acc_sh, sidxA, sidxB, didxA, didxB,
              rows0, rows1, gsem0, gsem1, isem):
    npad, d = acc_sh.shape
    c = lax.axis_index("c")
    s = lax.axis_index("s")
    rows_pt = npad // NS
    base_r = s * rows_pt
    rows = (rows0, rows1)
    gsems = (gsem0, gsem1)
    sidx = (sidxA, sidxB)
    didx = (didxA, didxB)
    T = steps // NSTAGE

    @pl.when(c == 0)
    def _():
        pltpu.sync_copy(g_hbm.at[pl.ds(base_r, rows_pt)], acc_sh.at[pl.ds(base_r, rows_pt)])
    @pl.when(c != 0)
    def _():
        pltpu.sync_copy(zeros_hbm.at[pl.ds(base_r, rows_pt)], acc_sh.at[pl.ds(base_r, rows_pt)])

    w = s * NC + c
    pltpu.sync_copy(src_hbm.at[w, 0], sidx[0])
    pltpu.sync_copy(dst_hbm.at[w, 0], didx[0])

    def fire_gather(q, r, b):
        pltpu.async_copy(g_hbm.at[sidx[q % 2].at[r]], rows[b], gsems[b])

    def wait_gather(b):
        pltpu.make_async_copy(g_hbm.at[pl.ds(0, EB)], rows[b], gsems[b]).wait()

    def scatter(q, r, b):
        pltpu.sync_copy(rows[b], acc_sh.at[didx[q % 2].at[r]], add=True)

    fire_gather(0, 0, 0)
    fire_gather(0, 1, 1)
    plsc.subcore_barrier()

    for q in range(NSTAGE):
        for j in range(4):
            b = j % 2
            wait_gather(b)
            scatter(q, j, b)
            fire_gather(q, j + 2, b)
        if q + 1 < NSTAGE:
            nb = (q + 1) % 2
            pltpu.async_copy(src_hbm.at[w, q + 1], sidx[nb], isem)
            pltpu.async_copy(dst_hbm.at[w, q + 1], didx[nb], isem)

        def body(i, carry):
            for pb in range(2):
                j = 2 * i + pb
                wait_gather(pb)
                scatter(q, j, pb)
                fire_gather(q, j + 2, pb)
            return carry
        lax.fori_loop(2, T // 2 - 1, body, 0)

        if q + 1 < NSTAGE:
            nb = (q + 1) % 2
            pltpu.make_async_copy(src_hbm.at[w, q + 1], sidx[nb], isem).wait()
            pltpu.make_async_copy(dst_hbm.at[w, q + 1], didx[nb], isem).wait()
        for r, j in enumerate(range(T - 2, T)):
            b = j % 2
            wait_gather(b)
            scatter(q, j, b)
            if q + 1 < NSTAGE:
                fire_gather(q + 1, r, b)

    plsc.subcore_barrier()
    pltpu.sync_copy(acc_sh.at[pl.ds(base_r, rows_pt)],
                    out_hbm.at[c, pl.ds(base_r, rows_pt)])


def _dinv_body(degs_ref, out_ref):
    d = degs_ref[0, :] + degs_ref[1, :] + 1.0
    out_ref[...] = lax.rsqrt(d)


def _mm_first_body(x_ref, w_ref, dinv_ref, g_ref):
    h = jnp.dot(x_ref[...], w_ref[...], preferred_element_type=jnp.float32)
    g_ref[...] = h * dinv_ref[...]


def _mm_mid_body(p0_ref, p1_ref, b_ref, dinv_ref, w_ref, g_ref):
    z = jnp.maximum((p0_ref[...] + p1_ref[...]) * dinv_ref[...] + b_ref[...], 0.0)
    h = jnp.dot(z, w_ref[...], preferred_element_type=jnp.float32)
    g_ref[...] = h * dinv_ref[...]


def _mm_last_body(p0_ref, p1_ref, b_ref, dinv_ref, out_ref):
    out_ref[...] = (p0_ref[...] + p1_ref[...]) * dinv_ref[...] + b_ref[...]


def kernel(x, edge_index, W1, b1, W2, b2, W3, b3):
    n, d = x.shape
    e = edge_index.shape[1]
    npad = _ceil_to(n + 1, 2048)
    epad = _ceil_to(e, NW * EB * NSTAGE * 2)
    steps = epad // (NW * EB)
    R = 2048

    src = jnp.concatenate([edge_index[0], jnp.zeros((epad - e,), jnp.int32)])
    dst = jnp.concatenate([edge_index[1], jnp.full((epad - e,), n, jnp.int32)])
    dst_flat = dst.reshape(NW, steps, EB)
    src = src.reshape(NW, NSTAGE, steps // NSTAGE, EB)
    dst = dst.reshape(NW, NSTAGE, steps // NSTAGE, EB)
    xp = jnp.pad(x, ((0, npad - n), (0, 0)))
    zeros2d = jnp.zeros((npad, d), jnp.float32)
    zeros1d = jnp.zeros((npad,), jnp.float32)

    mesh = plsc.VectorSubcoreMesh(core_axis_name="c", subcore_axis_name="s",
                                  num_cores=NC, num_subcores=NS)

    deg_call = pl.kernel(
        functools.partial(_deg_body, steps),
        out_type=jax.ShapeDtypeStruct((NC, npad), jnp.float32),
        mesh=mesh,
        scratch_types=[
            pltpu.VMEM_SHARED((npad,), jnp.float32),
            pltpu.VMEM((steps, EB), jnp.int32),
            pltpu.VMEM((EB,), jnp.float32),
            pltpu.SemaphoreType.DMA,
        ],
    )

    agg_call = pl.kernel(
        functools.partial(_agg_body, steps),
        out_type=jax.ShapeDtypeStruct((NC, npad, d), jnp.float32),
        mesh=mesh,
        scratch_types=[
            pltpu.VMEM_SHARED((npad, d), jnp.float32),
            pltpu.VMEM((steps // NSTAGE, EB), jnp.int32),
            pltpu.VMEM((steps // NSTAGE, EB), jnp.int32),
            pltpu.VMEM((steps // NSTAGE, EB), jnp.int32),
            pltpu.VMEM((steps // NSTAGE, EB), jnp.int32),
            pltpu.VMEM((EB, d), jnp.float32),
            pltpu.VMEM((EB, d), jnp.float32),
            pltpu.SemaphoreType.DMA,
            pltpu.SemaphoreType.DMA,
            pltpu.SemaphoreType.DMA,
        ],
    )

    dinv_call = pl.pallas_call(
        _dinv_body,
        out_shape=jax.ShapeDtypeStruct((npad,), jnp.float32),
    )

    row_spec = pl.BlockSpec((R, d), lambda i: (i, 0))
    w_spec = pl.BlockSpec((d, d), lambda i: (0, 0))
    b_spec = pl.BlockSpec((1, d), lambda i: (0, 0))
    dinv_spec = pl.BlockSpec((R, 1), lambda i: (i, 0))
    grid = (npad // R,)
    gshape = jax.ShapeDtypeStruct((npad, d), jnp.float32)

    mm_first = pl.pallas_call(
        _mm_first_body, grid=grid,
        in_specs=[row_spec, w_spec, dinv_spec],
        out_specs=row_spec, out_shape=gshape,
    )
    mm_mid = pl.pallas_call(
        _mm_mid_body, grid=grid,
        in_specs=[row_spec, row_spec, b_spec, dinv_spec, w_spec],
        out_specs=row_spec, out_shape=gshape,
    )
    mm_last = pl.pallas_call(
        _mm_last_body, grid=grid,
        in_specs=[row_spec, row_spec, b_spec, dinv_spec],
        out_specs=row_spec, out_shape=gshape,
    )

    degs = deg_call(dst_flat, zeros1d)
    dinv_col = dinv_call(degs).reshape(npad, 1)
    b1r = b1.reshape(1, d)
    b2r = b2.reshape(1, d)
    b3r = b3.reshape(1, d)

    g = mm_first(xp, W1, dinv_col)
    p = agg_call(g, src, dst, zeros2d)
    g = mm_mid(p[0], p[1], b1r, dinv_col, W2)
    p = agg_call(g, src, dst, zeros2d)
    g = mm_mid(p[0], p[1], b2r, dinv_col, W3)
    p = agg_call(g, src, dst, zeros2d)
    out = mm_last(p[0], p[1], b3r, dinv_col)
    return out[:n]


# speedup vs baseline: 2.1830x; 2.1728x over previous
import functools

import jax
import jax.numpy as jnp
from jax import lax
from jax.experimental import pallas as pl
from jax.experimental.pallas import tpu as pltpu
from jax.experimental.pallas import tpu_sc as plsc

NC = 2
NS = 16
NW = NC * NS
EB = 128
NSTAGE = 4


def _ceil_to(a, m):
    return (a + m - 1) // m * m


def _deg_body(steps, dst_hbm, zeros_hbm, out_hbm, deg_sh, didx_v, ones_v, sem):
    npad = deg_sh.shape[0]
    c = lax.axis_index("c")
    s = lax.axis_index("s")
    rows = npad // NS
    base_r = s * rows
    pltpu.sync_copy(zeros_hbm.at[pl.ds(base_r, rows)], deg_sh.at[pl.ds(base_r, rows)])
    def fill(k, carry):
        ones_v[pl.ds(k * 16, 16)] = jnp.ones((16,), jnp.float32)
        return carry
    lax.fori_loop(0, EB // 16, fill, 0)
    w = s * NC + c
    pltpu.sync_copy(dst_hbm.at[w], didx_v)
    plsc.subcore_barrier()

    def body(i, carry):
        for b in range(4):
            pltpu.async_copy(ones_v, deg_sh.at[didx_v.at[i * 4 + b]],
                             sem, add=True)
        for b in range(4):
            pltpu.make_async_copy(ones_v, deg_sh.at[didx_v.at[i * 4 + b]],
                                  sem).wait()
        return carry
    lax.fori_loop(0, steps // 4, body, 0)
    plsc.subcore_barrier()
    pltpu.sync_copy(deg_sh.at[pl.ds(base_r, rows)], out_hbm.at[c, pl.ds(base_r, rows)])


def _exp_body(steps, g_hbm, src_hbm, out_hbm,
              table_sh, sidxA, sidxB, rows0, rows1, gsem0, gsem1, isem):
    npad, d = table_sh.shape
    c = lax.axis_index("c")
    s = lax.axis_index("s")
    rows_pt = npad // NS
    base_r = s * rows_pt
    rows = (rows0, rows1)
    gsems = (gsem0, gsem1)
    sidx = (sidxA, sidxB)
    T = steps // NSTAGE

    pltpu.sync_copy(g_hbm.at[pl.ds(base_r, rows_pt)], table_sh.at[pl.ds(base_r, rows_pt)])
    w = s * NC + c
    pltpu.sync_copy(src_hbm.at[w, 0], sidx[0])
    plsc.subcore_barrier()

    def fire_gather(q, r, b):
        pltpu.async_copy(table_sh.at[sidx[q % 2].at[r]], rows[b], gsems[b])

    def wait_gather(b):
        pltpu.make_async_copy(g_hbm.at[pl.ds(0, EB)], rows[b], gsems[b]).wait()

    def write(q, r, b):
        pltpu.sync_copy(rows[b], out_hbm.at[w, q * T + r])

    fire_gather(0, 0, 0)
    fire_gather(0, 1, 1)

    for q in range(NSTAGE):
        for j in range(4):
            b = j % 2
            wait_gather(b)
            write(q, j, b)
            fire_gather(q, j + 2, b)
        if q + 1 < NSTAGE:
            nb = (q + 1) % 2
            pltpu.async_copy(src_hbm.at[w, q + 1], sidx[nb], isem)

        def body(i, carry):
            for pb in range(2):
                j = 2 * i + pb
                wait_gather(pb)
                write(q, j, pb)
                fire_gather(q, j + 2, pb)
            return carry
        lax.fori_loop(2, T // 2 - 1, body, 0)

        if q + 1 < NSTAGE:
            nb = (q + 1) % 2
            pltpu.make_async_copy(src_hbm.at[w, q + 1], sidx[nb], isem).wait()
        for r, j in enumerate(range(T - 2, T)):
            b = j % 2
            wait_gather(b)
            write(q, j, b)
            if q + 1 < NSTAGE:
                fire_gather(q + 1, r, b)


def _scat_body(steps, exp_hbm, dst_hbm, g_hbm, zeros_hbm, out_hbm,
               acc_sh, didxA, didxB, rows0, rows1, rsem0, rsem1, isem):
    npad, d = acc_sh.shape
    c = lax.axis_index("c")
    s = lax.axis_index("s")
    rows_pt = npad // NS
    base_r = s * rows_pt
    rows = (rows0, rows1)
    rsems = (rsem0, rsem1)
    didx = (didxA, didxB)
    T = steps // NSTAGE

    @pl.when(c == 0)
    def _():
        pltpu.sync_copy(g_hbm.at[pl.ds(base_r, rows_pt)], acc_sh.at[pl.ds(base_r, rows_pt)])
    @pl.when(c != 0)
    def _():
        pltpu.sync_copy(zeros_hbm.at[pl.ds(base_r, rows_pt)], acc_sh.at[pl.ds(base_r, rows_pt)])

    w = s * NC + c
    pltpu.sync_copy(dst_hbm.at[w, 0], didx[0])

    def fire_read(q, r, b):
        pltpu.async_copy(exp_hbm.at[w, q * T + r], rows[b], rsems[b])

    def wait_read(b):
        pltpu.make_async_copy(exp_hbm.at[w, 0], rows[b], rsems[b]).wait()

    def scatter(q, r, b):
        pltpu.sync_copy(rows[b], acc_sh.at[didx[q % 2].at[r]], add=True)

    fire_read(0, 0, 0)
    fire_read(0, 1, 1)
    plsc.subcore_barrier()

    for q in range(NSTAGE):
        for j in range(4):
            b = j % 2
            wait_read(b)
            scatter(q, j, b)
            fire_read(q, j + 2, b)
        if q + 1 < NSTAGE:
            nb = (q + 1) % 2
            pltpu.async_copy(dst_hbm.at[w, q + 1], didx[nb], isem)

        def body(i, carry):
            for pb in range(2):
                j = 2 * i + pb
                wait_read(pb)
                scatter(q, j, pb)
                fire_read(q, j + 2, pb)
            return carry
        lax.fori_loop(2, T // 2 - 1, body, 0)

        if q + 1 < NSTAGE:
            nb = (q + 1) % 2
            pltpu.make_async_copy(dst_hbm.at[w, q + 1], didx[nb], isem).wait()
        for r, j in enumerate(range(T - 2, T)):
            b = j % 2
            wait_read(b)
            scatter(q, j, b)
            if q + 1 < NSTAGE:
                fire_read(q + 1, r, b)

    plsc.subcore_barrier()
    pltpu.sync_copy(acc_sh.at[pl.ds(base_r, rows_pt)],
                    out_hbm.at[c, pl.ds(base_r, rows_pt)])


def _dinv_body(degs_ref, out_ref):
    d = degs_ref[0, :] + degs_ref[1, :] + 1.0
    out_ref[...] = lax.rsqrt(d)


def _mm_first_body(x_ref, w_ref, dinv_ref, g_ref):
    h = jnp.dot(x_ref[...], w_ref[...], preferred_element_type=jnp.float32)
    g_ref[...] = h * dinv_ref[...]


def _mm_mid_body(p0_ref, p1_ref, b_ref, dinv_ref, w_ref, g_ref):
    z = jnp.maximum((p0_ref[...] + p1_ref[...]) * dinv_ref[...] + b_ref[...], 0.0)
    h = jnp.dot(z, w_ref[...], preferred_element_type=jnp.float32)
    g_ref[...] = h * dinv_ref[...]


def _mm_last_body(p0_ref, p1_ref, b_ref, dinv_ref, out_ref):
    out_ref[...] = (p0_ref[...] + p1_ref[...]) * dinv_ref[...] + b_ref[...]


def kernel(x, edge_index, W1, b1, W2, b2, W3, b3):
    n, d = x.shape
    e = edge_index.shape[1]
    npad = _ceil_to(n + 1, 2048)
    epad = _ceil_to(e, NW * EB * NSTAGE * 2)
    steps = epad // (NW * EB)
    R = 2048

    src = jnp.concatenate([edge_index[0], jnp.zeros((epad - e,), jnp.int32)])
    dst = jnp.concatenate([edge_index[1], jnp.full((epad - e,), n, jnp.int32)])
    dst_flat = dst.reshape(NW, steps, EB)
    src = src.reshape(NW, NSTAGE, steps // NSTAGE, EB)
    dst = dst.reshape(NW, NSTAGE, steps // NSTAGE, EB)
    xp = jnp.pad(x, ((0, npad - n), (0, 0)))
    zeros2d = jnp.zeros((npad, d), jnp.float32)
    zeros1d = jnp.zeros((npad,), jnp.float32)

    mesh = plsc.VectorSubcoreMesh(core_axis_name="c", subcore_axis_name="s",
                                  num_cores=NC, num_subcores=NS)

    deg_call = pl.kernel(
        functools.partial(_deg_body, steps),
        out_type=jax.ShapeDtypeStruct((NC, npad), jnp.float32),
        mesh=mesh,
        scratch_types=[
            pltpu.VMEM_SHARED((npad,), jnp.float32),
            pltpu.VMEM((steps, EB), jnp.int32),
            pltpu.VMEM((EB,), jnp.float32),
            pltpu.SemaphoreType.DMA,
        ],
    )

    exp_call = pl.kernel(
        functools.partial(_exp_body, steps),
        out_type=jax.ShapeDtypeStruct((NW, steps, EB, d), jnp.float32),
        mesh=mesh,
        scratch_types=[
            pltpu.VMEM_SHARED((npad, d), jnp.float32),
            pltpu.VMEM((steps // NSTAGE, EB), jnp.int32),
            pltpu.VMEM((steps // NSTAGE, EB), jnp.int32),
            pltpu.VMEM((EB, d), jnp.float32),
            pltpu.VMEM((EB, d), jnp.float32),
            pltpu.SemaphoreType.DMA,
            pltpu.SemaphoreType.DMA,
            pltpu.SemaphoreType.DMA,
        ],
    )

    scat_call = pl.kernel(
        functools.partial(_scat_body, steps),
        out_type=jax.ShapeDtypeStruct((NC, npad, d), jnp.float32),
        mesh=mesh,
        scratch_types=[
            pltpu.VMEM_SHARED((npad, d), jnp.float32),
            pltpu.VMEM((steps // NSTAGE, EB), jnp.int32),
            pltpu.VMEM((steps // NSTAGE, EB), jnp.int32),
            pltpu.VMEM((EB, d), jnp.float32),
            pltpu.VMEM((EB, d), jnp.float32),
            pltpu.SemaphoreType.DMA,
            pltpu.SemaphoreType.DMA,
            pltpu.SemaphoreType.DMA,
        ],
    )

    dinv_call = pl.pallas_call(
        _dinv_body,
        out_shape=jax.ShapeDtypeStruct((npad,), jnp.float32),
    )

    row_spec = pl.BlockSpec((R, d), lambda i: (i, 0))
    w_spec = pl.BlockSpec((d, d), lambda i: (0, 0))
    b_spec = pl.BlockSpec((1, d), lambda i: (0, 0))
    dinv_spec = pl.BlockSpec((R, 1), lambda i: (i, 0))
    grid = (npad // R,)
    gshape = jax.ShapeDtypeStruct((npad, d), jnp.float32)

    mm_first = pl.pallas_call(
        _mm_first_body, grid=grid,
        in_specs=[row_spec, w_spec, dinv_spec],
        out_specs=row_spec, out_shape=gshape,
    )
    mm_mid = pl.pallas_call(
        _mm_mid_body, grid=grid,
        in_specs=[row_spec, row_spec, b_spec, dinv_spec, w_spec],
        out_specs=row_spec, out_shape=gshape,
    )
    mm_last = pl.pallas_call(
        _mm_last_body, grid=grid,
        in_specs=[row_spec, row_spec, b_spec, dinv_spec],
        out_specs=row_spec, out_shape=gshape,
    )

    degs = deg_call(dst_flat, zeros1d)
    dinv_col = dinv_call(degs).reshape(npad, 1)
    b1r = b1.reshape(1, d)
    b2r = b2.reshape(1, d)
    b3r = b3.reshape(1, d)

    g = mm_first(xp, W1, dinv_col)
    p = scat_call(exp_call(g, src), dst, g, zeros2d)
    g = mm_mid(p[0], p[1], b1r, dinv_col, W2)
    p = scat_call(exp_call(g, src), dst, g, zeros2d)
    g = mm_mid(p[0], p[1], b2r, dinv_col, W3)
    p = scat_call(exp_call(g, src), dst, g, zeros2d)
    out = mm_last(p[0], p[1], b3r, dinv_col)
    return out[:n]
